# baseline (speedup 1.0000x reference)
import functools

import jax
import jax.numpy as jnp
from jax import lax
from jax.experimental import pallas as pl
from jax.experimental.pallas import tpu as pltpu
from jax.experimental.pallas import tpu_sc as plsc


def _conv(x, w, b, stride, pad):
    out = lax.conv_general_dilated(
        x, w, (stride, stride), ((pad, pad), (pad, pad)),
        dimension_numbers=('NCHW', 'OIHW', 'NCHW'))
    return out + b.reshape(1, -1, 1, 1)


def _conv_t(x, w, b, stride, pad):
    k = w.shape[2]
    w2 = jnp.flip(w, (2, 3)).transpose(1, 0, 2, 3)
    p = k - 1 - pad
    out = lax.conv_general_dilated(
        x, w2, (1, 1), ((p, p), (p, p)), lhs_dilation=(stride, stride),
        dimension_numbers=('NCHW', 'OIHW', 'NCHW'))
    return out + b.reshape(1, -1, 1, 1)


def _bn(x):
    m = x.mean(axis=(0, 2, 3), keepdims=True)
    v = x.var(axis=(0, 2, 3), keepdims=True)
    return (x - m) / jnp.sqrt(v + 1e-5)


def _vq_argmin_body(z_ref, w_ref, idx_ref):
    z = z_ref[...]
    K = w_ref.shape[0]

    def dist_to(k):
        wk = w_ref[k, :]
        diff = z - wk[None, None, :]
        return jnp.sum(diff * diff, axis=2)

    def step(k, carry):
        mind, arg = carry
        dk = dist_to(k)
        upd = dk < mind
        return jnp.where(upd, dk, mind), jnp.where(upd, k, arg)

    mind0 = dist_to(0)
    arg0 = jnp.zeros(mind0.shape, jnp.int32)
    _, arg = lax.fori_loop(1, K, step, (mind0, arg0))
    idx_ref[...] = arg


def _vq_argmin(z3, vq_w):
    R = z3.shape[0]
    return pl.pallas_call(
        _vq_argmin_body,
        out_shape=jax.ShapeDtypeStruct((R, 128), jnp.int32),
    )(z3, vq_w)


def _sc_gather_counts(vq_w, idx3, n_real):
    NW, NCHUNK, CH = idx3.shape
    K, D = vq_w.shape
    BPW = NCHUNK * CH
    B = NW * BPW
    NC = plsc.get_sparse_core_info().num_cores

    mesh = plsc.VectorSubcoreMesh(core_axis_name="c", subcore_axis_name="s")

    @functools.partial(
        pl.kernel,
        mesh=mesh,
        compiler_params=pltpu.CompilerParams(needs_layout_passes=False),
        out_type=[jax.ShapeDtypeStruct((B, D), jnp.float32),
                  jax.ShapeDtypeStruct((NW, K), jnp.float32)],
        scratch_types=[
            pltpu.VMEM((NCHUNK, CH), jnp.int32),
            pltpu.VMEM((BPW, D), jnp.float32),
            pltpu.VMEM((K,), jnp.float32),
            pltpu.SemaphoreType.DMA,
        ],
    )
    def body(table_hbm, idx_hbm, rows_out, counts_out, idx_v, rows_v, cnt_v, gsem):
        wid = lax.axis_index("s") * NC + lax.axis_index("c")
        base = wid * BPW
        pltpu.sync_copy(idx_hbm.at[wid], idx_v)
        cps = [
            pltpu.async_copy(table_hbm.at[idx_v.at[c]],
                             rows_v.at[pl.ds(c * CH, CH)], gsem)
            for c in range(NCHUNK)
        ]
        for i in range(K // 16):
            cnt_v[pl.ds(i * 16, 16)] = jnp.zeros((16,), jnp.float32)
        ones = jnp.ones((16,), jnp.float32)
        for c in range(NCHUNK):
            for j in range(0, CH, 16):
                iv = idx_v[c, pl.ds(j, 16)]
                gpos = base + c * CH + j + lax.iota(jnp.int32, 16)
                plsc.addupdate_scatter(cnt_v, [iv], ones, mask=gpos < n_real)
        pltpu.sync_copy(cnt_v, counts_out.at[wid])
        for cp in cps:
            cp.wait()
        pltpu.sync_copy(rows_v, rows_out.at[pl.ds(base, BPW)])

    return body(vq_w, idx3)


def kernel(inputs, enc0_w, enc0_b, enc1_w, enc1_b, enc2_w, enc2_b, enc3_w,
           enc3_b, enc4_w, enc4_b, res0a_w, res0a_b, res0b_w, res0b_b,
           res1a_w, res1a_b, res1b_w, res1b_b, vq_w, dec0_w, dec0_b, dec1_w,
           dec1_b, dec2_w, dec2_b, dec3_w, dec3_b, channel_var):
    h = _conv(inputs, enc0_w, enc0_b, 1, 0)
    h = _conv(h, enc1_w, enc1_b, 2, 1); h = _bn(h); h = jax.nn.relu(h)
    h = _conv(h, enc2_w, enc2_b, 2, 1); h = _bn(h); h = jax.nn.relu(h)
    h = _conv(h, enc3_w, enc3_b, 2, 1); h = _bn(h); h = jax.nn.relu(h)
    h = _conv(h, enc4_w, enc4_b, 1, 1); h = _bn(h)
    for wa, ba, wb, bb in ((res0a_w, res0a_b, res0b_w, res0b_b),
                           (res1a_w, res1a_b, res1b_w, res1b_b)):
        r = jax.nn.relu(h)
        r = _conv(r, wa, ba, 1, 1); r = _bn(r); r = jax.nn.relu(r)
        r = _conv(r, wb, bb, 1, 0); r = _bn(r)
        h = h + r
    z = h
    Bt, D, hh, ww = z.shape
    K = vq_w.shape[0]
    N = Bt * hh * ww

    NW = 32
    CH = 112
    NCHUNK = -(-N // (NW * CH))
    B = NW * NCHUNK * CH
    zf = z.transpose(0, 2, 3, 1).reshape(N, D)
    zf_pad = jnp.concatenate(
        [zf, jnp.zeros((B - N, D), jnp.float32)], axis=0)
    idx2d = _vq_argmin(zf_pad.reshape(B // 128, 128, D), vq_w)
    idx3 = idx2d.reshape(NW, NCHUNK, CH)
    rows, counts_pw = _sc_gather_counts(vq_w, idx3, N)
    quantized = rows[:N].reshape(Bt, hh, ww, D).transpose(0, 3, 1, 2)
    zq = z + (quantized - z)
    e_latent = jnp.mean((quantized - z) ** 2)
    q_latent = jnp.mean((quantized - z) ** 2)
    c_loss = q_latent + 0.25 * e_latent
    avg_probs = counts_pw.sum(axis=0) / N
    perplexity = jnp.exp(-jnp.sum(avg_probs * jnp.log(avg_probs + 1e-10)))

    d = _conv_t(zq, dec0_w, dec0_b, 2, 1); d = jax.nn.relu(d)
    d = _conv_t(d, dec1_w, dec1_b, 2, 1); d = jax.nn.relu(d)
    d = _conv_t(d, dec2_w, dec2_b, 2, 1); d = jax.nn.relu(d)
    decoded = _conv(d, dec3_w, dec3_b, 1, 0)
    recon_loss = jnp.mean(((decoded - inputs) ** 2) / channel_var)
    total_loss = recon_loss + c_loss
    return decoded, recon_loss, c_loss, perplexity, total_loss
